# baseline (speedup 1.0000x reference)
import jax
import jax.numpy as jnp
from jax.experimental import pallas as pl
from jax.experimental.pallas import tpu as pltpu

_NBANDS = 8
_NBUF = 8
_AHEAD = 8


def _stream_body(x_hbm, w_ref, b_ref, o_hbm, in_buf, out_buf, in_sems, out_sems):
    d = w_ref.shape[0]
    w = w_ref[...]
    b = b_ref[...]

    def in_copy(i):
        return pltpu.make_async_copy(
            x_hbm.at[pl.ds(i * d, d), :], in_buf.at[i % _NBUF], in_sems.at[i % _NBUF]
        )

    def out_copy(i):
        return pltpu.make_async_copy(
            out_buf.at[i % _NBUF], o_hbm.at[pl.ds(i * d, d), :], out_sems.at[i % _NBUF]
        )

    for i in range(_AHEAD):
        in_copy(i).start()
    for i in range(_NBANDS):
        in_copy(i).wait()
        xb = in_buf[i % _NBUF].astype(jnp.bfloat16)
        out_buf[i % _NBUF] = (
            jax.lax.dot_general(
                w, xb, (((1,), (0,)), ((), ())),
                preferred_element_type=jnp.float32,
            )
            + b
        )
        out_copy(i).start()
    for i in range(_NBANDS):
        out_copy(i).wait()


def kernel(x, edge_index, edge_attr, semantic_data, W_out, b_out):
    n, t, f, d = x.shape
    xt = jnp.transpose(x, (1, 2, 3, 0)).reshape(t * f * d, n)
    wb = W_out.astype(jnp.bfloat16)
    b2 = b_out.reshape(d, 1)
    out = pl.pallas_call(
        _stream_body,
        in_specs=[
            pl.BlockSpec(memory_space=pltpu.MemorySpace.HBM),
            pl.BlockSpec(memory_space=pltpu.MemorySpace.VMEM),
            pl.BlockSpec(memory_space=pltpu.MemorySpace.VMEM),
        ],
        out_specs=pl.BlockSpec(memory_space=pltpu.MemorySpace.HBM),
        out_shape=jax.ShapeDtypeStruct((t * f * d, n), jnp.float32),
        scratch_shapes=[
            pltpu.VMEM((_NBUF, d, n), jnp.float32),
            pltpu.VMEM((_NBUF, d, n), jnp.float32),
            pltpu.SemaphoreType.DMA((_NBUF,)),
            pltpu.SemaphoreType.DMA((_NBUF,)),
        ],
    )(xt, wb, b2)
    return jnp.transpose(out.reshape(t, f, d, n), (3, 0, 1, 2))


# speedup vs baseline: 1.0108x; 1.0108x over previous
import jax
import jax.numpy as jnp
from jax.experimental import pallas as pl
from jax.experimental.pallas import tpu as pltpu

_NBANDS = 8
_NBUF = 4
_AHEAD = 3


def _stream_body(x_hbm, w_ref, b_ref, o_hbm, in_buf, out_buf, in_sems, out_sems):
    d = w_ref.shape[0]
    w = w_ref[...]
    b = b_ref[...]

    def in_copy(i):
        return pltpu.make_async_copy(
            x_hbm.at[pl.ds(i * d, d), :], in_buf.at[i % _NBUF], in_sems.at[i % _NBUF]
        )

    def out_copy(i):
        return pltpu.make_async_copy(
            out_buf.at[i % _NBUF], o_hbm.at[pl.ds(i * d, d), :], out_sems.at[i % _NBUF]
        )

    for i in range(_AHEAD):
        in_copy(i).start()
    for i in range(_NBANDS):
        if i + _AHEAD < _NBANDS:
            in_copy(i + _AHEAD).start()
        in_copy(i).wait()
        if i >= _NBUF:
            out_copy(i - _NBUF).wait()
        xb = in_buf[i % _NBUF].astype(jnp.bfloat16)
        out_buf[i % _NBUF] = (
            jax.lax.dot_general(
                w, xb, (((1,), (0,)), ((), ())),
                preferred_element_type=jnp.float32,
            )
            + b
        )
        out_copy(i).start()
    for i in range(_NBANDS - _NBUF, _NBANDS):
        out_copy(i).wait()


def kernel(x, edge_index, edge_attr, semantic_data, W_out, b_out):
    n, t, f, d = x.shape
    xt = jnp.transpose(x, (1, 2, 3, 0)).reshape(t * f * d, n)
    wb = W_out.astype(jnp.bfloat16)
    b2 = b_out.reshape(d, 1)
    out = pl.pallas_call(
        _stream_body,
        in_specs=[
            pl.BlockSpec(memory_space=pltpu.MemorySpace.HBM),
            pl.BlockSpec(memory_space=pltpu.MemorySpace.VMEM),
            pl.BlockSpec(memory_space=pltpu.MemorySpace.VMEM),
        ],
        out_specs=pl.BlockSpec(memory_space=pltpu.MemorySpace.HBM),
        out_shape=jax.ShapeDtypeStruct((t * f * d, n), jnp.float32),
        scratch_shapes=[
            pltpu.VMEM((_NBUF, d, n), jnp.float32),
            pltpu.VMEM((_NBUF, d, n), jnp.float32),
            pltpu.SemaphoreType.DMA((_NBUF,)),
            pltpu.SemaphoreType.DMA((_NBUF,)),
        ],
    )(xt, wb, b2)
    return jnp.transpose(out.reshape(t, f, d, n), (3, 0, 1, 2))
